# baseline (speedup 1.0000x reference)
import functools
import jax
import jax.numpy as jnp
from jax import lax
from jax.experimental import pallas as pl
from jax.experimental.pallas import tpu as pltpu
from jax.experimental.pallas import tpu_sc as plsc


def _tc_add_body(x_ref, e_ref, o_ref):
    o_ref[...] = x_ref[...] + e_ref[...]


def _tc_add(inputs, pos, SBLK=2048, ST=None):
    B, S, D = inputs.shape
    if ST is None:
        ST = S
    n_sblk = ST // SBLK
    return pl.pallas_call(
        _tc_add_body,
        grid=(n_sblk, B),
        in_specs=[
            pl.BlockSpec((1, SBLK, D), lambda s, b: (b, s, 0)),
            pl.BlockSpec((SBLK, D), lambda s, b: (s, 0)),
        ],
        out_specs=pl.BlockSpec((1, SBLK, D), lambda s, b: (b, s, 0)),
        out_shape=jax.ShapeDtypeStruct((B, S, D), inputs.dtype),
        compiler_params=pltpu.CompilerParams(skip_device_barrier=True),
    )(inputs, pos)


def _make_sc_add(B, S, D, s0=0, CH=16):
    info = plsc.get_sparse_core_info()
    NC = info.num_cores
    NW = NC * info.num_subcores
    SC_S = S - s0
    RW = SC_S // NW
    NCH = RW // CH
    NV = D // info.num_lanes
    mesh = plsc.VectorSubcoreMesh(core_axis_name="c", subcore_axis_name="s")

    def body(x_hbm, e_hbm, o_hbm,
             ev0, ev1, xv0, xv1, xv2, xv3,
             se0, se1, sx0, sx1, sx2, sx3, so0, so1, so2, so3):
        wid = lax.axis_index("s") * NC + lax.axis_index("c")
        obase = wid * RW
        base = s0 + obase
        evs, ses = (ev0, ev1), (se0, se1)
        xvs = (xv0, xv1, xv2, xv3)
        sxs = (sx0, sx1, sx2, sx3)
        sos = (so0, so1, so2, so3)

        def e_src(c):
            return e_hbm.at[pl.ds(base + c * CH, CH)]

        def x_src(step):
            c, b = divmod(step, B)
            return x_hbm.at[b, pl.ds(base + c * CH, CH)]

        def o_dst(step):
            c, b = divmod(step, B)
            return o_hbm.at[b, pl.ds(obase + c * CH, CH)]

        NSTEP = NCH * B
        pltpu.async_copy(e_src(0), ev0, se0)
        pltpu.async_copy(x_src(0), xv0, sx0)
        pltpu.async_copy(x_src(1), xv1, sx1)
        for step in range(NSTEP):
            c, b = divmod(step, B)
            if b == 0:
                if c + 1 < NCH:
                    pltpu.async_copy(e_src(c + 1), evs[(c + 1) % 2],
                                     ses[(c + 1) % 2])
                pltpu.make_async_copy(e_src(c), evs[c % 2], ses[c % 2]).wait()
            if step >= 2:
                pltpu.make_async_copy(xvs[(step - 2) % 4], o_dst(step - 2),
                                      sos[(step - 2) % 4]).wait()
            if step + 2 < NSTEP:
                pltpu.async_copy(x_src(step + 2), xvs[(step + 2) % 4],
                                 sxs[(step + 2) % 4])
            pltpu.make_async_copy(x_src(step), xvs[step % 4],
                                  sxs[step % 4]).wait()
            xv, ev = xvs[step % 4], evs[c % 2]

            @plsc.parallel_loop(0, CH * NV, 1, unroll=8)
            def cbody(i):
                r = i // NV
                col = (i % NV) * 16
                plsc.addupdate(xv.at[r, pl.ds(col, 16)],
                               ev[r, pl.ds(col, 16)])

            pltpu.async_copy(xv, o_dst(step), sos[step % 4])
        pltpu.make_async_copy(xvs[(NSTEP - 2) % 4], o_dst(NSTEP - 2),
                              sos[(NSTEP - 2) % 4]).wait()
        pltpu.make_async_copy(xvs[(NSTEP - 1) % 4], o_dst(NSTEP - 1),
                              sos[(NSTEP - 1) % 4]).wait()

    vm = lambda: pltpu.VMEM((CH, D), jnp.float32)
    return pl.kernel(
        body,
        mesh=mesh,
        out_type=jax.ShapeDtypeStruct((B, SC_S, D), jnp.float32),
        scratch_types=[vm(), vm(), vm(), vm(), vm(), vm()]
        + [pltpu.SemaphoreType.DMA] * 10,
    )


def kernel(inputs, embeddings):
    B, S, D = inputs.shape
    pos = embeddings[:S]
    ST = 6144
    sc_part = _make_sc_add(B, S, D, s0=ST)(inputs, pos)
    tc_part = inputs[:, :ST] + pos[:ST][None]
    return jnp.concatenate([tc_part, sc_part], axis=1)


# speedup vs baseline: 1.5777x; 1.5777x over previous
import functools
import jax
import jax.numpy as jnp
from jax import lax
from jax.experimental import pallas as pl
from jax.experimental.pallas import tpu as pltpu
from jax.experimental.pallas import tpu_sc as plsc


def _tc_add_body(x_ref, e_ref, o_ref):
    o_ref[...] = x_ref[...] + e_ref[...]


def _tc_add(inputs, pos, SBLK=2048, ST=None):
    B, S, D = inputs.shape
    if ST is None:
        ST = S
    n_sblk = ST // SBLK
    return pl.pallas_call(
        _tc_add_body,
        grid=(n_sblk, B),
        in_specs=[
            pl.BlockSpec((1, SBLK, D), lambda s, b: (b, s, 0)),
            pl.BlockSpec((SBLK, D), lambda s, b: (s, 0)),
        ],
        out_specs=pl.BlockSpec((1, SBLK, D), lambda s, b: (b, s, 0)),
        out_shape=jax.ShapeDtypeStruct((B, S, D), inputs.dtype),
        compiler_params=pltpu.CompilerParams(skip_device_barrier=True),
    )(inputs, pos)


def _make_sc_add(B, S, D, s0=0, CH=8, RING=8, PRE=4):
    info = plsc.get_sparse_core_info()
    NC = info.num_cores
    NW = NC * info.num_subcores
    SC_S = S - s0
    RW = SC_S // NW
    NCH = RW // CH
    NV = D // info.num_lanes
    mesh = plsc.VectorSubcoreMesh(core_axis_name="c", subcore_axis_name="s")

    def body(x_hbm, e_hbm, o_hbm, *scratch):
        evs = scratch[:2]
        xvs = scratch[2:2 + RING]
        ses = scratch[2 + RING:4 + RING]
        sxs = scratch[4 + RING:4 + 2 * RING]
        sos = scratch[4 + 2 * RING:4 + 3 * RING]
        wid = lax.axis_index("s") * NC + lax.axis_index("c")
        obase = wid * RW
        base = s0 + obase

        def e_src(c):
            return e_hbm.at[pl.ds(base + c * CH, CH)]

        def x_src(step):
            c, b = divmod(step, B)
            return x_hbm.at[b, pl.ds(base + c * CH, CH)]

        def o_dst(step):
            c, b = divmod(step, B)
            return o_hbm.at[b, pl.ds(obase + c * CH, CH)]

        NSTEP = NCH * B
        drained = set()
        pltpu.async_copy(e_src(0), evs[0], ses[0])
        for p in range(min(PRE, NSTEP)):
            pltpu.async_copy(x_src(p), xvs[p % RING], sxs[p % RING])
        for step in range(NSTEP):
            c, b = divmod(step, B)
            if b == 0:
                if c + 1 < NCH:
                    pltpu.async_copy(e_src(c + 1), evs[(c + 1) % 2],
                                     ses[(c + 1) % 2])
                pltpu.make_async_copy(e_src(c), evs[c % 2], ses[c % 2]).wait()
            if step + PRE < NSTEP:
                prev = step + PRE - RING
                if prev >= 0:
                    pltpu.make_async_copy(xvs[prev % RING], o_dst(prev),
                                          sos[prev % RING]).wait()
                    drained.add(prev)
                pltpu.async_copy(x_src(step + PRE), xvs[(step + PRE) % RING],
                                 sxs[(step + PRE) % RING])
            pltpu.make_async_copy(x_src(step), xvs[step % RING],
                                  sxs[step % RING]).wait()
            xv, ev = xvs[step % RING], evs[c % 2]

            @plsc.parallel_loop(0, CH * NV, 1, unroll=8)
            def cbody(i):
                r = i // NV
                col = (i % NV) * 16
                plsc.addupdate(xv.at[r, pl.ds(col, 16)],
                               ev[r, pl.ds(col, 16)])

            pltpu.async_copy(xv, o_dst(step), sos[step % RING])
        for step in range(NSTEP):
            if step not in drained:
                pltpu.make_async_copy(xvs[step % RING], o_dst(step),
                                      sos[step % RING]).wait()

    vm = lambda: pltpu.VMEM((CH, D), jnp.float32)
    return pl.kernel(
        body,
        mesh=mesh,
        out_type=jax.ShapeDtypeStruct((B, SC_S, D), jnp.float32),
        scratch_types=[vm(), vm()] + [vm() for _ in range(RING)]
        + [pltpu.SemaphoreType.DMA] * (2 + 2 * RING),
    )


def kernel(inputs, embeddings):
    B, S, D = inputs.shape
    pos = embeddings[:S]
    return _make_sc_add(B, S, D)(inputs, pos)


# speedup vs baseline: 2.2674x; 1.4372x over previous
import functools
import jax
import jax.numpy as jnp
from jax import lax
from jax.experimental import pallas as pl
from jax.experimental.pallas import tpu as pltpu
from jax.experimental.pallas import tpu_sc as plsc


def _tc_add_body(x_ref, e_ref, o_ref):
    o_ref[...] = x_ref[...] + e_ref[...]


def _tc_add(inputs, pos, SBLK=2048, ST=None):
    B, S, D = inputs.shape
    if ST is None:
        ST = S
    n_sblk = ST // SBLK
    return pl.pallas_call(
        _tc_add_body,
        grid=(n_sblk, B),
        in_specs=[
            pl.BlockSpec((1, SBLK, D), lambda s, b: (b, s, 0)),
            pl.BlockSpec((SBLK, D), lambda s, b: (s, 0)),
        ],
        out_specs=pl.BlockSpec((1, SBLK, D), lambda s, b: (b, s, 0)),
        out_shape=jax.ShapeDtypeStruct((B, S, D), inputs.dtype),
    )(inputs, pos)


def _make_sc_add(B, S, D, s0=0, CH=8, RING=8, PRE=4):
    info = plsc.get_sparse_core_info()
    NC = info.num_cores
    NW = NC * info.num_subcores
    SC_S = S - s0
    RW = SC_S // NW
    NCH = RW // CH
    NV = D // info.num_lanes
    mesh = plsc.VectorSubcoreMesh(core_axis_name="c", subcore_axis_name="s")

    def body(x_hbm, e_hbm, o_hbm, *scratch):
        evs = scratch[:2]
        xvs = scratch[2:2 + RING]
        ses = scratch[2 + RING:4 + RING]
        sxs = scratch[4 + RING:4 + 2 * RING]
        sos = scratch[4 + 2 * RING:4 + 3 * RING]
        wid = lax.axis_index("s") * NC + lax.axis_index("c")
        obase = wid * RW
        base = s0 + obase

        def e_src(c):
            return e_hbm.at[pl.ds(base + c * CH, CH)]

        def x_src(step):
            c, b = divmod(step, B)
            return x_hbm.at[b, pl.ds(base + c * CH, CH)]

        def o_dst(step):
            c, b = divmod(step, B)
            return o_hbm.at[b, pl.ds(obase + c * CH, CH)]

        NSTEP = NCH * B
        drained = set()
        pltpu.async_copy(e_src(0), evs[0], ses[0])
        for p in range(min(PRE, NSTEP)):
            pltpu.async_copy(x_src(p), xvs[p % RING], sxs[p % RING])
        for step in range(NSTEP):
            c, b = divmod(step, B)
            if b == 0:
                if c + 1 < NCH:
                    pltpu.async_copy(e_src(c + 1), evs[(c + 1) % 2],
                                     ses[(c + 1) % 2])
                pltpu.make_async_copy(e_src(c), evs[c % 2], ses[c % 2]).wait()
            if step + PRE < NSTEP:
                prev = step + PRE - RING
                if prev >= 0:
                    pltpu.make_async_copy(xvs[prev % RING], o_dst(prev),
                                          sos[prev % RING]).wait()
                    drained.add(prev)
                pltpu.async_copy(x_src(step + PRE), xvs[(step + PRE) % RING],
                                 sxs[(step + PRE) % RING])
            pltpu.make_async_copy(x_src(step), xvs[step % RING],
                                  sxs[step % RING]).wait()
            xv, ev = xvs[step % RING], evs[c % 2]

            @plsc.parallel_loop(0, CH * NV, 1, unroll=8)
            def cbody(i):
                r = i // NV
                col = (i % NV) * 16
                plsc.addupdate(xv.at[r, pl.ds(col, 16)],
                               ev[r, pl.ds(col, 16)])

            pltpu.async_copy(xv, o_dst(step), sos[step % RING])
        for step in range(NSTEP):
            if step not in drained:
                pltpu.make_async_copy(xvs[step % RING], o_dst(step),
                                      sos[step % RING]).wait()

    vm = lambda: pltpu.VMEM((CH, D), jnp.float32)
    return pl.kernel(
        body,
        mesh=mesh,
        out_type=jax.ShapeDtypeStruct((B, SC_S, D), jnp.float32),
        scratch_types=[vm(), vm()] + [vm() for _ in range(RING)]
        + [pltpu.SemaphoreType.DMA] * (2 + 2 * RING),
    )


def kernel(inputs, embeddings):
    B, S, D = inputs.shape
    pos = embeddings[:S]
    return _tc_add(inputs, pos)
